# Initial kernel scaffold; baseline (speedup 1.0000x reference)
import jax
import jax.numpy as jnp
from jax.experimental import pallas as pl


def kernel(text_ids, table):
    raise NotImplementedError("write your pallas kernel here")


# speedup vs baseline: 3.3416x; 3.3416x over previous
import functools

import jax
import jax.numpy as jnp
from jax import lax
from jax.experimental import pallas as pl
from jax.experimental.pallas import tpu as pltpu
from jax.experimental.pallas import tpu_sc as plsc

D = 128
B_ROWS = 4096
B_COLS = 50
B_TOTAL = B_ROWS * B_COLS

_info = plsc.get_sparse_core_info()
NC = _info.num_cores
NS = _info.num_subcores
NW = NC * NS
B_PER_W = B_TOTAL // NW
CHUNK = 128
NCHUNK = B_PER_W // CHUNK
NBUF = 5

_mesh = plsc.VectorSubcoreMesh(core_axis_name="c", subcore_axis_name="s")


@functools.partial(
    pl.kernel,
    mesh=_mesh,
    out_type=jax.ShapeDtypeStruct((B_TOTAL, D), jnp.float32),
    scratch_types=[
        pltpu.VMEM((NCHUNK, CHUNK), jnp.int32),
        pltpu.VMEM((NBUF, CHUNK, D), jnp.float32),
        pltpu.SemaphoreType.DMA,
        pltpu.SemaphoreType.DMA,
    ],
)
def _embed_gather(ids_hbm, table_hbm, out_hbm, idx_v, rows_v, gsem, wsem):
    wid = lax.axis_index("s") * NC + lax.axis_index("c")
    base = wid * B_PER_W

    pltpu.sync_copy(ids_hbm.at[wid], idx_v)

    for b in range(NBUF):
        pltpu.async_copy(table_hbm.at[idx_v.at[b]], rows_v.at[b], gsem)

    def outer(g, carry):
        for b in range(NBUF):
            j = g * NBUF + b
            pltpu.make_async_copy(
                table_hbm.at[idx_v.at[j]], rows_v.at[b], gsem
            ).wait()
            wb = pltpu.make_async_copy(
                rows_v.at[b],
                out_hbm.at[pl.ds(base + j * CHUNK, CHUNK)],
                wsem,
            )
            wb.start()

            @pl.when(j + NBUF < NCHUNK)
            def _():
                wb.wait()
                pltpu.async_copy(
                    table_hbm.at[idx_v.at[j + NBUF]], rows_v.at[b], gsem
                )
        return carry

    lax.fori_loop(0, NCHUNK // NBUF, outer, 0)

    for b in range(NBUF):
        j = NCHUNK - NBUF + b
        pltpu.make_async_copy(
            rows_v.at[b],
            out_hbm.at[pl.ds(base + j * CHUNK, CHUNK)],
            wsem,
        ).wait()


def kernel(text_ids, table):
    ids = text_ids.reshape(NW, NCHUNK, CHUNK).astype(jnp.int32)
    out = _embed_gather(ids, table)
    return out.reshape(B_ROWS, B_COLS, D)


# speedup vs baseline: 6.0198x; 1.8015x over previous
import functools

import jax
import jax.numpy as jnp
from jax import lax
from jax.experimental import pallas as pl
from jax.experimental.pallas import tpu as pltpu
from jax.experimental.pallas import tpu_sc as plsc

D = 128
B_ROWS = 4096
B_COLS = 50
COLS_PAD = 56

_info = plsc.get_sparse_core_info()
NC = _info.num_cores
NS = _info.num_subcores
NW = NC * NS
ROWS_PER_W = B_ROWS // NW
R = 4
NGROUP = ROWS_PER_W // R
NBUF = 4

_mesh = plsc.VectorSubcoreMesh(core_axis_name="c", subcore_axis_name="s")


@functools.partial(
    pl.kernel,
    mesh=_mesh,
    out_type=jax.ShapeDtypeStruct((B_ROWS, B_COLS, D), jnp.float32),
    scratch_types=[
        pltpu.VMEM((ROWS_PER_W, COLS_PAD), jnp.int32),
        pltpu.VMEM((NBUF, R, B_COLS, D), jnp.float32),
        pltpu.SemaphoreType.DMA,
        pltpu.SemaphoreType.DMA,
    ],
)
def _embed_gather(ids_hbm, table_hbm, out_hbm, idx_v, rows_v, gsem, wsem):
    wid = lax.axis_index("s") * NC + lax.axis_index("c")
    row0 = wid * ROWS_PER_W

    pltpu.sync_copy(ids_hbm.at[wid], idx_v)

    def start_group(g, b):
        for rr in range(R):
            pltpu.async_copy(
                table_hbm.at[idx_v.at[g * R + rr, pl.ds(0, B_COLS)]],
                rows_v.at[b, rr],
                gsem,
            )

    def wait_group(g, b):
        for rr in range(R):
            pltpu.make_async_copy(
                table_hbm.at[idx_v.at[g * R + rr, pl.ds(0, B_COLS)]],
                rows_v.at[b, rr],
                gsem,
            ).wait()

    for b in range(NBUF):
        start_group(b, b)

    def outer(o, carry):
        for b in range(NBUF):
            g = o * NBUF + b
            wait_group(g, b)
            wb = pltpu.make_async_copy(
                rows_v.at[b],
                out_hbm.at[pl.ds(row0 + g * R, R)],
                wsem,
            )
            wb.start()

            @pl.when(g + NBUF < NGROUP)
            def _():
                wb.wait()
                start_group(g + NBUF, b)
        return carry

    lax.fori_loop(0, NGROUP // NBUF, outer, 0)

    for b in range(NBUF):
        g = NGROUP - NBUF + b
        pltpu.make_async_copy(
            rows_v.at[b],
            out_hbm.at[pl.ds(row0 + g * R, R)],
            wsem,
        ).wait()


def kernel(text_ids, table):
    ids = jnp.pad(text_ids.astype(jnp.int32), ((0, 0), (0, COLS_PAD - B_COLS)))
    ids = ids.reshape(NW, ROWS_PER_W, COLS_PAD)
    return _embed_gather(ids, table)


# speedup vs baseline: 10.6837x; 1.7748x over previous
import functools

import jax
import jax.numpy as jnp
from jax import lax
from jax.experimental import pallas as pl
from jax.experimental.pallas import tpu as pltpu
from jax.experimental.pallas import tpu_sc as plsc

D = 128
B_ROWS = 4096
B_COLS = 50

_info = plsc.get_sparse_core_info()
NC = _info.num_cores
NS = _info.num_subcores
NW = NC * NS
SPAN = B_ROWS // NW
NBUF = 5

_mesh = plsc.VectorSubcoreMesh(core_axis_name="c", subcore_axis_name="s")


@functools.partial(
    pl.kernel,
    mesh=_mesh,
    out_type=jax.ShapeDtypeStruct((B_COLS, B_ROWS, D), jnp.float32),
    scratch_types=[
        pltpu.VMEM((B_COLS, SPAN), jnp.int32),
        pltpu.VMEM((NBUF, SPAN, D), jnp.float32),
        pltpu.SemaphoreType.DMA,
        pltpu.SemaphoreType.DMA,
    ],
)
def _embed_gather(ids_hbm, table_hbm, out_hbm, idx_v, rows_v, gsem, wsem):
    wid = lax.axis_index("s") * NC + lax.axis_index("c")
    i0 = wid * SPAN

    pltpu.sync_copy(ids_hbm.at[:, pl.ds(i0, SPAN)], idx_v)

    for b in range(NBUF):
        pltpu.async_copy(table_hbm.at[idx_v.at[b]], rows_v.at[b], gsem)

    def outer(g, carry):
        for b in range(NBUF):
            j = g * NBUF + b
            pltpu.make_async_copy(
                table_hbm.at[idx_v.at[j]], rows_v.at[b], gsem
            ).wait()
            wb = pltpu.make_async_copy(
                rows_v.at[b],
                out_hbm.at[j, pl.ds(i0, SPAN)],
                wsem,
            )
            wb.start()

            @pl.when(j + NBUF < B_COLS)
            def _():
                wb.wait()
                pltpu.async_copy(
                    table_hbm.at[idx_v.at[j + NBUF]], rows_v.at[b], gsem
                )
        return carry

    lax.fori_loop(0, B_COLS // NBUF, outer, 0)

    for b in range(NBUF):
        j = B_COLS - NBUF + b
        pltpu.make_async_copy(
            rows_v.at[b],
            out_hbm.at[j, pl.ds(i0, SPAN)],
            wsem,
        ).wait()


def kernel(text_ids, table):
    ids_t = text_ids.astype(jnp.int32).T
    out_t = _embed_gather(ids_t, table)
    return out_t.transpose(1, 0, 2)
